# baseline (speedup 1.0000x reference)
import functools

import jax
import jax.numpy as jnp
import numpy as np
from jax.experimental import pallas as pl
from jax.experimental.pallas import tpu as pltpu

EPS = 1e-6
THETA = 1000000.0


def _rms_qkv_body(h_ref, ln_ref, w_ref, o_ref):
    x = h_ref[...]
    v = jnp.mean(x * x, axis=1, keepdims=True)
    xn = x * jax.lax.rsqrt(v + EPS) * ln_ref[...]
    o_ref[...] = jnp.dot(xn.astype(jnp.bfloat16), w_ref[...],
                         preferred_element_type=jnp.float32)


def _rms_qkv(hidden2d, ln1_w, wqkv_t, bt):
    s, d = hidden2d.shape
    nqkv = wqkv_t.shape[1]
    return pl.pallas_call(
        _rms_qkv_body,
        grid=(s // bt,),
        in_specs=[
            pl.BlockSpec((bt, d), lambda i: (i, 0)),
            pl.BlockSpec((1, d), lambda i: (0, 0)),
            pl.BlockSpec((d, nqkv), lambda i: (0, 0)),
        ],
        out_specs=pl.BlockSpec((bt, nqkv), lambda i: (i, 0)),
        out_shape=jax.ShapeDtypeStruct((s, nqkv), jnp.float32),
        compiler_params=pltpu.CompilerParams(
            dimension_semantics=("parallel",)),
    )(hidden2d, ln1_w.reshape(1, d), wqkv_t)


def _attn_body(q_ref, k_ref, v_ref, o_ref, *, bq, s, dh, rscale):
    i = pl.program_id(1)
    q = q_ref[0]
    k = k_ref[0]
    v = v_ref[0]
    scores = jax.lax.dot_general(
        q, k, (((1,), (1,)), ((), ())),
        preferred_element_type=jnp.float32) * rscale
    qpos = i * bq + jax.lax.broadcasted_iota(jnp.int32, (bq, s), 0)
    kpos = jax.lax.broadcasted_iota(jnp.int32, (bq, s), 1)
    scores = jnp.where(qpos >= kpos, scores, jnp.float32(-1e9))
    m = jnp.max(scores, axis=1, keepdims=True)
    p = jnp.exp(scores - m)
    p = p / jnp.sum(p, axis=1, keepdims=True)
    o_ref[0] = jnp.dot(p.astype(jnp.bfloat16), v,
                       preferred_element_type=jnp.float32)


def _attention(q, k, v, bq):
    h, s, dh = q.shape
    kvh = k.shape[0]
    rep = h // kvh
    body = functools.partial(_attn_body, bq=bq, s=s, dh=dh,
                             rscale=1.0 / float(np.sqrt(dh)))
    return pl.pallas_call(
        body,
        grid=(h, s // bq),
        in_specs=[
            pl.BlockSpec((1, bq, dh), lambda hh, i: (hh, i, 0)),
            pl.BlockSpec((1, s, dh), lambda hh, i: (hh // rep, 0, 0)),
            pl.BlockSpec((1, s, dh), lambda hh, i: (hh // rep, 0, 0)),
        ],
        out_specs=pl.BlockSpec((1, bq, dh), lambda hh, i: (hh, i, 0)),
        out_shape=jax.ShapeDtypeStruct((h, s, dh), jnp.float32),
        compiler_params=pltpu.CompilerParams(
            dimension_semantics=("parallel", "parallel")),
    )(q, k, v)


def _oproj_body(ctx_ref, ow_ref, h_ref, ln_ref, gw_ref, h2_ref, xn_ref,
                gl_ref):
    h2 = h_ref[...] + jnp.dot(ctx_ref[...], ow_ref[...],
                              preferred_element_type=jnp.float32)
    v = jnp.mean(h2 * h2, axis=1, keepdims=True)
    xn = h2 * jax.lax.rsqrt(v + EPS) * ln_ref[...]
    h2_ref[...] = h2
    xn_ref[...] = xn.astype(jnp.bfloat16)
    gl_ref[...] = jnp.dot(xn, gw_ref[...],
                          preferred_element_type=jnp.float32,
                          precision=jax.lax.Precision.HIGHEST)


def _oproj_rms_gate(ctx2d, ow_t, hidden2d, ln2_w, gate_t, bt):
    s, d = hidden2d.shape
    e = gate_t.shape[1]
    return pl.pallas_call(
        _oproj_body,
        grid=(s // bt,),
        in_specs=[
            pl.BlockSpec((bt, d), lambda i: (i, 0)),
            pl.BlockSpec((d, d), lambda i: (0, 0)),
            pl.BlockSpec((bt, d), lambda i: (i, 0)),
            pl.BlockSpec((1, d), lambda i: (0, 0)),
            pl.BlockSpec((d, e), lambda i: (0, 0)),
        ],
        out_specs=[
            pl.BlockSpec((bt, d), lambda i: (i, 0)),
            pl.BlockSpec((bt, d), lambda i: (i, 0)),
            pl.BlockSpec((bt, e), lambda i: (i, 0)),
        ],
        out_shape=[
            jax.ShapeDtypeStruct((s, d), jnp.float32),
            jax.ShapeDtypeStruct((s, d), jnp.bfloat16),
            jax.ShapeDtypeStruct((s, e), jnp.float32),
        ],
        compiler_params=pltpu.CompilerParams(
            dimension_semantics=("parallel",)),
    )(ctx2d, ow_t, hidden2d, ln2_w.reshape(1, d), gate_t)


def _moe_body(x_ref, w1_ref, w3_ref, w2_ref, c_ref, hres_ref, o_ref, *,
              n_e):
    e = pl.program_id(1)
    x = x_ref[...]
    h1 = jax.lax.dot_general(x, w1_ref[0], (((1,), (1,)), ((), ())),
                             preferred_element_type=jnp.float32)
    h3 = jax.lax.dot_general(x, w3_ref[0], (((1,), (1,)), ((), ())),
                             preferred_element_type=jnp.float32)
    g = (jax.nn.silu(h1) * h3).astype(jnp.bfloat16)
    out_e = jax.lax.dot_general(g, w2_ref[0], (((1,), (1,)), ((), ())),
                                preferred_element_type=jnp.float32)
    eids = jax.lax.broadcasted_iota(jnp.int32, c_ref.shape, 1)
    w = jnp.sum(jnp.where(eids == e, c_ref[...], 0.0), axis=1,
                keepdims=True)
    contrib = out_e * w

    @pl.when(e == 0)
    def _():
        o_ref[...] = hres_ref[...] + contrib

    @pl.when(e > 0)
    def _():
        o_ref[...] += contrib


def _moe(xn2, w1, w3, w2, combine, hres, bt):
    s, d = hres.shape
    n_e, ff, _ = w1.shape
    body = functools.partial(_moe_body, n_e=n_e)
    return pl.pallas_call(
        body,
        grid=(s // bt, n_e),
        in_specs=[
            pl.BlockSpec((bt, d), lambda t, e: (t, 0)),
            pl.BlockSpec((1, ff, d), lambda t, e: (e, 0, 0)),
            pl.BlockSpec((1, ff, d), lambda t, e: (e, 0, 0)),
            pl.BlockSpec((1, d, ff), lambda t, e: (e, 0, 0)),
            pl.BlockSpec((bt, n_e), lambda t, e: (t, 0)),
            pl.BlockSpec((bt, d), lambda t, e: (t, 0)),
        ],
        out_specs=pl.BlockSpec((bt, d), lambda t, e: (t, 0)),
        out_shape=jax.ShapeDtypeStruct((s, d), jnp.float32),
        compiler_params=pltpu.CompilerParams(
            dimension_semantics=("parallel", "arbitrary")),
    )(xn2, w1, w3, w2, combine, hres)


def _rotate_half(x):
    h = x.shape[-1] // 2
    return jnp.concatenate([-x[..., h:], x[..., :h]], axis=-1)


def kernel(hidden_states, position_ids, lb_loss, ln1_w, q_w, k_w, v_w,
           o_w, ln2_w, gate_w, W1, W2, W3):
    b, s, d = hidden_states.shape
    n_e, ff, _ = W1.shape
    dh = 64
    h = q_w.shape[0] // dh
    kvh = k_w.shape[0] // dh
    topk = 2
    bt = 256 if s % 256 == 0 else s
    bq = bt

    hidden2d = hidden_states.reshape(s, d)
    wqkv_t = jnp.concatenate([q_w, k_w, v_w], axis=0).T.astype(jnp.bfloat16)
    qkv = _rms_qkv(hidden2d, ln1_w, wqkv_t, bt)

    q = qkv[:, : h * dh].reshape(s, h, dh).transpose(1, 0, 2)
    k = qkv[:, h * dh: (h + kvh) * dh].reshape(s, kvh, dh).transpose(1, 0, 2)
    v = qkv[:, (h + kvh) * dh:].reshape(s, kvh, dh).transpose(1, 0, 2)

    inv_freq = 1.0 / (THETA ** (np.arange(0, dh, 2, dtype=np.float32) / dh))
    freqs = position_ids.reshape(s).astype(jnp.float32)[:, None] * inv_freq[None, :]
    emb = jnp.concatenate([freqs, freqs], axis=-1)
    cos = jnp.cos(emb)[None, :, :]
    sin = jnp.sin(emb)[None, :, :]
    q = (q * cos + _rotate_half(q) * sin).astype(jnp.bfloat16)
    k = (k * cos + _rotate_half(k) * sin).astype(jnp.bfloat16)
    v = v.astype(jnp.bfloat16)

    ctx = _attention(q, k, v, bq)
    ctx2d = ctx.transpose(1, 0, 2).reshape(s, h * dh).astype(jnp.bfloat16)

    hres, xn2, glogits = _oproj_rms_gate(
        ctx2d, o_w.T.astype(jnp.bfloat16), hidden2d, ln2_w,
        gate_w.T.astype(jnp.float32), bt)

    probs = jax.nn.softmax(glogits, axis=-1)
    rw, sel = jax.lax.top_k(probs, topk)
    rwn = rw / jnp.sum(rw, axis=-1, keepdims=True)
    combine = jnp.sum(
        jax.nn.one_hot(sel, n_e, dtype=jnp.float32) * rwn[..., None], axis=1)
    tokens_per_expert = jnp.mean(
        jax.nn.one_hot(sel, n_e, dtype=jnp.float32), axis=0)
    router_prob = jnp.mean(probs, axis=0)[None, :]
    lb = jnp.mean(jnp.sum(tokens_per_expert * router_prob, axis=-1)) * n_e

    out2d = _moe(xn2, W1.astype(jnp.bfloat16), W3.astype(jnp.bfloat16),
                 W2.astype(jnp.bfloat16), combine, hres, bt)

    return out2d.reshape(b, s, d), position_ids, lb_loss + lb
